# baseline (speedup 1.0000x reference)
import jax
import jax.numpy as jnp
from jax import lax
from jax.experimental import pallas as pl
from jax.experimental.pallas import tpu as pltpu
from jax.experimental.pallas import tpu_sc as plsc

B, L, F = 4096, 20, 7
O_LIN, O_EMB, O = 16, 8, 24
N = B * L
NC, NS, LANES = 2, 16, 16
NW = NC * NS
EPT = N // NW
XW = EPT * F
OW = EPT * O
G = 4
OUTER = EPT // (LANES * G)


def _sc_body(x_hbm, m_hbm, wb_hbm, br_hbm, tab_hbm, out_hbm,
             x_v, m_v, wb_v, br_v, tab_v, o_v):
    wid = lax.axis_index("s") * NC + lax.axis_index("c")
    pltpu.sync_copy(x_hbm.at[pl.ds(wid * XW, XW)], x_v)
    pltpu.sync_copy(m_hbm.at[pl.ds(wid * EPT, EPT)], m_v)
    pltpu.sync_copy(wb_hbm, wb_v)
    pltpu.sync_copy(br_hbm, br_v)
    pltpu.sync_copy(tab_hbm, tab_v)
    iota = lax.iota(jnp.int32, LANES)

    def body(i, carry):
        base = i * (LANES * G)
        xm, mv, mapped, idx24 = [], [], [], []
        for g in range(G):
            idxe = iota + (base + g * LANES)
            a7 = idxe * F
            m = m_v[pl.ds(base + g * LANES, LANES)]
            feats = [plsc.load_gather(x_v, [a7 + f]) for f in range(6)]
            xm.append([feats[f] * m for f in range(5)])
            z = (feats[5] * m).astype(jnp.int32)
            mapped.append(jnp.where((z >= 57) & (z <= 80), z - 56, 0))
            mv.append(m)
            idx24.append(idxe * O)
        for o in range(O_LIN):
            bo = br_v[pl.ds(o * LANES, LANES)]
            w = [wb_v[pl.ds((o * 5 + f) * LANES, LANES)] for f in range(5)]
            for g in range(G):
                acc = bo
                for f in range(5):
                    acc = acc + xm[g][f] * w[f]
                plsc.store_scatter(o_v, [idx24[g] + o], jnp.maximum(acc, 0.0))
        for g in range(G):
            m8 = mapped[g] * O_EMB
            for j in range(O_EMB):
                e = plsc.load_gather(tab_v, [m8 + j])
                plsc.store_scatter(o_v, [idx24[g] + (O_LIN + j)], e * mv[g])
        return carry

    lax.fori_loop(0, OUTER, body, 0)
    pltpu.sync_copy(o_v, out_hbm.at[pl.ds(wid * OW, OW)])


def kernel(elements_info, elements_mask, W, b, tm_table):
    x_flat = elements_info.reshape(-1)
    m_flat = elements_mask.reshape(-1)
    wb = jnp.repeat(W.reshape(-1), LANES)
    br = jnp.repeat(b, LANES)
    tab = jnp.pad(tm_table.reshape(-1), (0, 56))
    mesh = plsc.VectorSubcoreMesh(core_axis_name="c", subcore_axis_name="s")
    out = pl.kernel(
        _sc_body,
        out_type=jax.ShapeDtypeStruct((N * O,), jnp.float32),
        mesh=mesh,
        compiler_params=pltpu.CompilerParams(needs_layout_passes=False),
        scratch_types=[
            pltpu.VMEM((XW,), jnp.float32),
            pltpu.VMEM((EPT,), jnp.float32),
            pltpu.VMEM((80 * LANES,), jnp.float32),
            pltpu.VMEM((O_LIN * LANES,), jnp.float32),
            pltpu.VMEM((256,), jnp.float32),
            pltpu.VMEM((OW,), jnp.float32),
        ],
    )(x_flat, m_flat, wb, br, tab)
    return out.reshape(B, L, O)


# speedup vs baseline: 16.2218x; 16.2218x over previous
import jax
import jax.numpy as jnp
from jax import lax
from jax.experimental import pallas as pl

B, L, F = 4096, 20, 7
O_LIN, O_EMB, O = 16, 8, 24
NTAB = 25
BK = 512


def _tc_body(x_ref, m_ref, w_ref, b_ref, t_ref, o_ref):
    w = w_ref[...]
    bvec = b_ref[...]
    tab = t_ref[...]
    for l in range(L):
        x = x_ref[l]
        m = m_ref[l][None, :]
        xm = x * m
        lin = lax.dot_general(
            w, xm[:5], (((0,), (0,)), ((), ())),
            preferred_element_type=jnp.float32,
        )
        lin = jax.nn.relu(lin + bvec)
        zi = xm[5:6].astype(jnp.int32)
        mapped = jnp.where((zi >= 57) & (zi <= 80), zi - 56, 0)
        kio = lax.broadcasted_iota(jnp.int32, (NTAB, BK), 0)
        onehot = (mapped == kio).astype(jnp.float32)
        emb = lax.dot_general(
            tab, onehot, (((1,), (0,)), ((), ())),
            preferred_element_type=jnp.float32,
        )
        o_ref[l, :O_LIN, :] = lin * m
        o_ref[l, O_LIN:, :] = emb * m


def kernel(elements_info, elements_mask, W, b, tm_table):
    x_t = jnp.transpose(elements_info, (1, 2, 0))
    m_t = jnp.transpose(elements_mask, (1, 0))
    w_t = jnp.transpose(W, (1, 0))
    t_t = jnp.transpose(tm_table, (1, 0))
    b2 = b[:, None]
    out = pl.pallas_call(
        _tc_body,
        grid=(B // BK,),
        in_specs=[
            pl.BlockSpec((L, F, BK), lambda i: (0, 0, i)),
            pl.BlockSpec((L, BK), lambda i: (0, i)),
            pl.BlockSpec((5, O_LIN), lambda i: (0, 0)),
            pl.BlockSpec((O_LIN, 1), lambda i: (0, 0)),
            pl.BlockSpec((O_EMB, NTAB), lambda i: (0, 0)),
        ],
        out_specs=pl.BlockSpec((L, O, BK), lambda i: (0, 0, i)),
        out_shape=jax.ShapeDtypeStruct((L, O, B), jnp.float32),
    )(x_t, m_t, w_t, b2, t_t)
    return jnp.transpose(out, (2, 0, 1))
